# Initial kernel scaffold; baseline (speedup 1.0000x reference)
import jax
import jax.numpy as jnp
from jax.experimental import pallas as pl


def kernel(prior_probs, edge_index, edge_probs):
    raise NotImplementedError("write your pallas kernel here")


# speedup vs baseline: 166.5463x; 166.5463x over previous
import functools

import jax
import jax.numpy as jnp
from jax import lax
from jax.experimental import pallas as pl
from jax.experimental.pallas import tpu as pltpu
from jax.experimental.pallas import tpu_sc as plsc

N_STEPS = 3
N = 100000
E = 6400000

NC = 2
NS = 16
NW = NC * NS

N_PAD = 100352
SL_SC = N_PAD // NS
SL_W = N_PAD // NW
SL_IT = SL_W // 16
NSUB = 4
SL_P = SL_SC // NSUB
SL_P_IT = SL_P // 16

ROWS = E // 64
ROWS_W = ROWS // NW
RPC = 25
CHUNKS = ROWS_W // RPC
C = RPC * 64

_mesh = plsc.VectorSubcoreMesh(core_axis_name="c", subcore_axis_name="s")
_f32 = jnp.float32
_params = pltpu.CompilerParams(use_tc_tiling_on_sc=False,
                               needs_layout_passes=False)


def _chain(prior_v, agg_list):
    p = prior_v
    prod = jnp.ones((16,), _f32)
    for agg in agg_list:
        delta = 1.0 - jnp.exp(-agg)
        new_p = prod * delta
        prod = prod * (1.0 - new_p)
        p = new_p
    return p, prod


def _phase1_compute(t, cid, sid, prior_hbm, partials, pbc, prior_sl, part, pbuf):
    for sub in range(NSUB):
        nb = sid * SL_SC + sub * SL_P
        pltpu.sync_copy(prior_hbm.at[pl.ds(nb, SL_P)], prior_sl)
        for i in range(t):
            pltpu.sync_copy(partials[i].at[0, pl.ds(nb, SL_P)],
                            part.at[pl.ds((2 * i) * SL_P, SL_P)])
            pltpu.sync_copy(partials[i].at[1, pl.ds(nb, SL_P)],
                            part.at[pl.ds((2 * i + 1) * SL_P, SL_P)])

        def body(k, _):
            o = k * 16
            aggs = [part[pl.ds((2 * i) * SL_P + o, 16)]
                    + part[pl.ds((2 * i + 1) * SL_P + o, 16)]
                    for i in range(t)]
            p, _prod = _chain(prior_sl[pl.ds(o, 16)], aggs)
            pbuf[pl.ds(o, 16)] = p
            return 0

        lax.fori_loop(0, SL_P_IT, body, 0)
        pltpu.sync_copy(pbuf, pbc.at[cid, pl.ds(nb, SL_P)])


def _make_step(t):
    n_in = 3 + t
    scratch = [
        pltpu.VMEM((N_PAD,), _f32),
        pltpu.VMEM((RPC, 64), jnp.int32),
        pltpu.VMEM((RPC, 64), jnp.int32),
        pltpu.VMEM((RPC, 64), _f32),
        pltpu.VMEM((RPC, 64), _f32),
        pltpu.VMEM((SL_P,), _f32),
        pltpu.VMEM((SL_P,), _f32),
        pltpu.VMEM((max(2 * t, 1) * SL_P,), _f32),
        pltpu.VMEM_SHARED((N_PAD,), _f32),
        pltpu.SemaphoreType.DMA,
    ]
    out_type = (
        jax.ShapeDtypeStruct((NC, N_PAD), _f32),
        jax.ShapeDtypeStruct((NC, N_PAD), _f32),
    )

    @functools.partial(pl.kernel, out_type=out_type, mesh=_mesh,
                       scratch_types=scratch, compiler_params=_params,
                       name=f"influence_step{t}")
    def step(*refs):
        (prior_hbm, edges3d, ep2d), partials = refs[:3], list(refs[3:n_in])
        partial_out, pbc = refs[n_in:n_in + 2]
        (p_tile, src2, tgt2, epb, msg, zbuf, pbuf, part,
         agg_sh, sem) = refs[n_in + 2:]
        cid = lax.axis_index("c")
        sid = lax.axis_index("s")
        wid = cid * NS + sid

        if t == 0:
            pltpu.sync_copy(prior_hbm, p_tile)
        else:
            _phase1_compute(t, cid, sid, prior_hbm, partials, pbc,
                            zbuf, part, pbuf)

        def zero_body(k, _):
            zbuf[pl.ds(k * 16, 16)] = jnp.zeros((16,), _f32)
            return 0
        lax.fori_loop(0, SL_P_IT, zero_body, 0)
        for sub in range(NSUB):
            pltpu.sync_copy(zbuf, agg_sh.at[pl.ds(sid * SL_SC + sub * SL_P, SL_P)])

        plsc.subcore_barrier()

        if t > 0:
            pltpu.sync_copy(pbc.at[cid], p_tile)

        def chunk_body(g, _):
            rb = wid * ROWS_W + g * RPC
            pltpu.sync_copy(edges3d.at[0, pl.ds(rb, RPC)], src2)
            pltpu.sync_copy(edges3d.at[1, pl.ds(rb, RPC)], tgt2)
            pltpu.sync_copy(ep2d.at[pl.ds(rb, RPC)], epb)
            for r in range(RPC):
                for l in range(4):
                    sl = pl.ds(l * 16, 16)
                    pv = plsc.load_gather(p_tile, [src2[r, sl]])
                    msg[r, sl] = epb[r, sl] * pv
            ds = [pltpu.async_copy(msg.at[j], agg_sh.at[tgt2.at[j]], sem,
                                   add=True)
                  for j in range(RPC)]
            for d in ds:
                d.wait()
            return 0
        lax.fori_loop(0, CHUNKS, chunk_body, 0)

        plsc.subcore_barrier()

        pltpu.sync_copy(agg_sh.at[pl.ds(sid * SL_SC, SL_SC)],
                        partial_out.at[cid, pl.ds(sid * SL_SC, SL_SC)])

    return step


def _make_final():
    scratch = [
        pltpu.VMEM((SL_W,), _f32),
        pltpu.VMEM((2 * N_STEPS * SL_W,), _f32),
        pltpu.VMEM((SL_W,), _f32),
    ]

    @functools.partial(pl.kernel,
                       out_type=jax.ShapeDtypeStruct((N_PAD,), _f32),
                       mesh=_mesh, scratch_types=scratch,
                       compiler_params=_params, name="influence_final")
    def final(prior_hbm, p0, p1, p2, out_hbm, prior_sl, part, obuf):
        cid = lax.axis_index("c")
        sid = lax.axis_index("s")
        wid = cid * NS + sid
        nb = wid * SL_W
        partials = (p0, p1, p2)
        pltpu.sync_copy(prior_hbm.at[pl.ds(nb, SL_W)], prior_sl)
        for i in range(N_STEPS):
            pltpu.sync_copy(partials[i].at[0, pl.ds(nb, SL_W)],
                            part.at[pl.ds((2 * i) * SL_W, SL_W)])
            pltpu.sync_copy(partials[i].at[1, pl.ds(nb, SL_W)],
                            part.at[pl.ds((2 * i + 1) * SL_W, SL_W)])

        def body(k, _):
            o = k * 16
            aggs = [part[pl.ds((2 * i) * SL_W + o, 16)]
                    + part[pl.ds((2 * i + 1) * SL_W + o, 16)]
                    for i in range(N_STEPS)]
            prior_v = prior_sl[pl.ds(o, 16)]
            _p, prod = _chain(prior_v, aggs)
            obuf[pl.ds(o, 16)] = 1.0 - prod + prior_v
            return 0

        lax.fori_loop(0, SL_IT, body, 0)
        pltpu.sync_copy(obuf, out_hbm.at[pl.ds(nb, SL_W)])

    return final


_steps = [_make_step(t) for t in range(N_STEPS)]
_final = _make_final()


def kernel(prior_probs, edge_index, edge_probs):
    prior_pad = jnp.zeros((N_PAD,), _f32).at[:N].set(prior_probs)
    edges3d = edge_index.astype(jnp.int32).reshape(2, ROWS, 64)
    ep2d = edge_probs.reshape(ROWS, 64)
    partials = []
    for t in range(N_STEPS):
        p_t, _pbc = _steps[t](prior_pad, edges3d, ep2d, *partials)
        partials.append(p_t)
    out_pad = _final(prior_pad, *partials)
    return out_pad[:N].reshape(-1, 1)
